# baseline (speedup 1.0000x reference)
import functools
import jax
import jax.numpy as jnp
from jax.experimental import pallas as pl

N = 10000
E = 320000
D = 128
DE = 16
UNITS = 128
EDGE_STEPS = 4
NODE_STEPS = 4

EB = 2000
EG = E // EB
NB = 1000
NG = N // NB


def _mm_bias_body(x_ref, w_ref, b_ref, o_ref):
    o_ref[...] = jnp.dot(x_ref[...], w_ref[...],
                         preferred_element_type=jnp.float32) + b_ref[...]


def _mm2_body(a_ref, b_ref, wa_ref, wb_ref, o_ref):
    o_ref[...] = (jnp.dot(a_ref[...], wa_ref[...], preferred_element_type=jnp.float32)
                  + jnp.dot(b_ref[...], wb_ref[...], preferred_element_type=jnp.float32))


def _sum2_mm_bias_body(p0_ref, p1_ref, w_ref, b_ref, o_ref):
    o_ref[...] = jnp.dot(p0_ref[...] + p1_ref[...], w_ref[...],
                         preferred_element_type=jnp.float32) + b_ref[...]


def _init_body(g_ref, ef_ref, wb_ref, b_ref, o_ref):
    o_ref[...] = jax.nn.relu(
        g_ref[...] + jnp.dot(ef_ref[...], wb_ref[...],
                             preferred_element_type=jnp.float32) + b_ref[...])


def _edge_step_fused_body(h0_ref, g_ref, hwrev_ref, wnext_ref, h_ref, hw_ref):
    h = jax.nn.relu(h0_ref[...] + g_ref[...] - hwrev_ref[...])
    h_ref[...] = h
    hw_ref[...] = jnp.dot(h, wnext_ref[...], preferred_element_type=jnp.float32)


def _edge_step_last_body(h0_ref, g_ref, hwrev_ref, h_ref):
    h_ref[...] = jax.nn.relu(h0_ref[...] + g_ref[...] - hwrev_ref[...])


def _node_step_fused_body(v_ref, a0_ref, a1_ref, b_ref, s_ref, wnext_ref, o_ref):
    x = jax.nn.relu(s_ref[0, 0] * v_ref[...] + a0_ref[...] + a1_ref[...] + b_ref[...])
    o_ref[...] = jnp.dot(x, wnext_ref[...], preferred_element_type=jnp.float32)


def _node_step_last_body(v_ref, a0_ref, a1_ref, b_ref, s_ref, o_ref):
    o_ref[...] = jax.nn.relu(
        s_ref[0, 0] * v_ref[...] + a0_ref[...] + a1_ref[...] + b_ref[...])


def _rowblk(B, ncols):
    return pl.BlockSpec((B, ncols), lambda i: (i, 0))


def _full(shape):
    return pl.BlockSpec(shape, lambda i: tuple(0 for _ in shape))


def _mm_bias(x, w, b, B):
    M, K = x.shape
    Nout = w.shape[1]
    return pl.pallas_call(
        _mm_bias_body,
        grid=(M // B,),
        in_specs=[_rowblk(B, K), _full((K, Nout)), _full((1, Nout))],
        out_specs=_rowblk(B, Nout),
        out_shape=jax.ShapeDtypeStruct((M, Nout), jnp.float32),
    )(x, w, b.reshape(1, -1))


def _mm2(a, b, wa, wb, B):
    M = a.shape[0]
    Nout = wa.shape[1]
    return pl.pallas_call(
        _mm2_body,
        grid=(M // B,),
        in_specs=[_rowblk(B, a.shape[1]), _rowblk(B, b.shape[1]),
                  _full(wa.shape), _full(wb.shape)],
        out_specs=_rowblk(B, Nout),
        out_shape=jax.ShapeDtypeStruct((M, Nout), jnp.float32),
    )(a, b, wa, wb)


def _sum2_mm_bias(p0, p1, w, b, B):
    M, K = p0.shape
    Nout = w.shape[1]
    return pl.pallas_call(
        _sum2_mm_bias_body,
        grid=(M // B,),
        in_specs=[_rowblk(B, K), _rowblk(B, K), _full((K, Nout)), _full((1, Nout))],
        out_specs=_rowblk(B, Nout),
        out_shape=jax.ShapeDtypeStruct((M, Nout), jnp.float32),
    )(p0, p1, w, b.reshape(1, -1))


def _edge_init(g, ef, wb, b):
    return pl.pallas_call(
        _init_body,
        grid=(EG,),
        in_specs=[_rowblk(EB, UNITS), _rowblk(EB, DE),
                  _full((DE, UNITS)), _full((1, UNITS))],
        out_specs=_rowblk(EB, UNITS),
        out_shape=jax.ShapeDtypeStruct((E, UNITS), jnp.float32),
    )(g, ef, wb, b.reshape(1, -1))


def _rev_spec():
    return pl.BlockSpec((EB, UNITS), lambda i: ((i + EG // 2) % EG, 0))


def _edge_step_fused(h0, g, hw, wnext):
    return pl.pallas_call(
        _edge_step_fused_body,
        grid=(EG,),
        in_specs=[_rowblk(EB, UNITS), _rowblk(EB, UNITS), _rev_spec(),
                  _full((UNITS, UNITS))],
        out_specs=[_rowblk(EB, UNITS), _rowblk(EB, UNITS)],
        out_shape=[jax.ShapeDtypeStruct((E, UNITS), jnp.float32),
                   jax.ShapeDtypeStruct((E, UNITS), jnp.float32)],
    )(h0, g, hw, wnext)


def _edge_step_last(h0, g, hw):
    return pl.pallas_call(
        _edge_step_last_body,
        grid=(EG,),
        in_specs=[_rowblk(EB, UNITS), _rowblk(EB, UNITS), _rev_spec()],
        out_specs=_rowblk(EB, UNITS),
        out_shape=jax.ShapeDtypeStruct((E, UNITS), jnp.float32),
    )(h0, g, hw)


def _node_step(v, a0, a1, b, scale, wnext):
    if wnext is None:
        body, extra_in, extra_specs = _node_step_last_body, (), ()
        nout = UNITS
    else:
        body = _node_step_fused_body
        extra_in = (wnext,)
        extra_specs = (_full((UNITS, UNITS)),)
        nout = UNITS
    return pl.pallas_call(
        body,
        grid=(NG,),
        in_specs=[_rowblk(NB, UNITS), _rowblk(NB, UNITS), _rowblk(NB, UNITS),
                  _full((1, UNITS)), _full((1, 1))] + list(extra_specs),
        out_specs=_rowblk(NB, nout),
        out_shape=jax.ShapeDtypeStruct((N, nout), jnp.float32),
    )(v, a0, a1, b.reshape(1, -1), scale.reshape(1, 1), *extra_in)


def _segsum(rows, dst):
    p = jax.ops.segment_sum(rows, dst, num_segments=N)
    z = jnp.zeros_like(p)
    return p, z


def _gather_rows(table, idx):
    return jnp.take(table, idx, axis=0)


def kernel(node_feature, edge_feature, edge_index, W_init, b_init,
           W_edge, b_edge, W_gin0, b_gin0, W_gin, b_gin, eps):
    src = edge_index[0].astype(jnp.int32)
    dst = edge_index[1].astype(jnp.int32)

    Wa = W_init[:D]
    Wb = W_init[D:]
    nfW = _mm_bias(node_feature, Wa, jnp.zeros((UNITS,), jnp.float32), NB)
    g0 = _gather_rows(nfW, src)
    h0 = _edge_init(g0, edge_feature, Wb, b_init)

    hW = _mm_bias(h0, W_edge[0], jnp.zeros((UNITS,), jnp.float32), EB)
    h = h0
    for t in range(EDGE_STEPS):
        p0, p1 = _segsum(h, dst)
        u = _sum2_mm_bias(p0, p1, W_edge[t], b_edge[t], NB)
        g = _gather_rows(u, src)
        if t < EDGE_STEPS - 1:
            h, hW = _edge_step_fused(h0, g, hW, W_edge[t + 1])
        else:
            h = _edge_step_last(h0, g, hW)

    m0, m1 = _segsum(h, dst)
    msg = m0 + m1
    W0a = W_gin0[:D]
    W0b = W_gin0[D:]
    v = _mm2(node_feature, msg, W0a, W0b, NB)
    for t in range(NODE_STEPS):
        vg = _gather_rows(v, src)
        a0, a1 = _segsum(vg, dst)
        b = b_gin0 if t == 0 else b_gin[t - 1]
        wnext = W_gin[t] if t < NODE_STEPS - 1 else None
        v = _node_step(v, a0, a1, b, 1.0 + eps[t], wnext)
    return v


# speedup vs baseline: 3.7814x; 3.7814x over previous
import functools
import jax
import jax.numpy as jnp
from jax import lax
from jax.experimental import pallas as pl
from jax.experimental.pallas import tpu as pltpu
from jax.experimental.pallas import tpu_sc as plsc

N = 10000
E = 320000
D = 128
DE = 16
UNITS = 128
EDGE_STEPS = 4
NODE_STEPS = 4

EB = 2000
EG = E // EB
NB = 1000
NG = N // NB

NC = 2
NS = 16
NW = NC * NS
EPW = E // NW
CH = 200
NCHUNK = EPW // CH
CHG = 400
NCHUNKG = EPW // CHG
NRC = N // CH
ACC_ROUNDS = -(-NRC // NS)

_sc_mesh = plsc.VectorSubcoreMesh(core_axis_name="c", subcore_axis_name="s")


def _zero_rowbuf(rowbuf, nrows):
    z = jnp.zeros((16,), jnp.float32)

    def body(r, _):
        for j in range(UNITS // 16):
            rowbuf[r, pl.ds(j * 16, 16)] = z
        return 0
    lax.fori_loop(0, nrows, body, 0)


def _zero_acc(acc, rowbuf, sid):
    _zero_rowbuf(rowbuf, CH)
    for r in range(ACC_ROUNDS):
        j = sid + r * NS

        @pl.when(j < NRC)
        def _():
            off = pl.multiple_of(j * CH, 8)
            pltpu.sync_copy(rowbuf, acc.at[pl.ds(off, CH)])


def _acc_to_out(acc, rowbuf, sid, out_slice):
    for r in range(ACC_ROUNDS):
        j = sid + r * NS

        @pl.when(j < NRC)
        def _():
            off = pl.multiple_of(j * CH, 8)
            pltpu.sync_copy(acc.at[pl.ds(off, CH)], rowbuf)
            pltpu.sync_copy(rowbuf, out_slice.at[pl.ds(off, CH)])


@functools.partial(
    pl.kernel,
    out_type=jax.ShapeDtypeStruct((NC, N, UNITS), jnp.float32),
    mesh=_sc_mesh,
    scratch_types=dict(
        idx_v=pltpu.VMEM((CH,), jnp.int32),
        rows_v=pltpu.VMEM((CH, UNITS), jnp.float32),
        acc=pltpu.VMEM_SHARED((N, UNITS), jnp.float32),
    ),
)
def _sc_segsum_rows(rows_hbm, dst_hbm, out_hbm, idx_v, rows_v, acc):
    cid = lax.axis_index("c")
    sid = lax.axis_index("s")
    wid = sid * NC + cid
    _zero_acc(acc, rows_v, sid)
    plsc.subcore_barrier()
    base = wid * EPW

    def body(k, _):
        off = pl.multiple_of(base + k * CH, 8)
        pltpu.sync_copy(dst_hbm.at[pl.ds(off, CH)], idx_v)
        pltpu.sync_copy(rows_hbm.at[pl.ds(off, CH)], rows_v)
        pltpu.sync_copy(rows_v, acc.at[idx_v], add=True)
        return 0
    lax.fori_loop(0, NCHUNK, body, 0)
    plsc.subcore_barrier()
    _acc_to_out(acc, rows_v, sid, out_hbm.at[cid])


@functools.partial(
    pl.kernel,
    out_type=jax.ShapeDtypeStruct((E, UNITS), jnp.float32),
    mesh=_sc_mesh,
    scratch_types=dict(
        idx_v=pltpu.VMEM((CHG,), jnp.int32),
        rows_v=pltpu.VMEM((CHG, UNITS), jnp.float32),
        sem=pltpu.SemaphoreType.DMA,
    ),
)
def _sc_gather_rows(table_hbm, idx_hbm, out_hbm, idx_v, rows_v, sem):
    cid = lax.axis_index("c")
    sid = lax.axis_index("s")
    wid = sid * NC + cid
    base = wid * EPW

    def body(k, _):
        off = pl.multiple_of(base + k * CHG, 8)
        pltpu.sync_copy(idx_hbm.at[pl.ds(off, CHG)], idx_v)
        pltpu.async_copy(table_hbm.at[idx_v], rows_v, sem).wait()
        pltpu.sync_copy(rows_v, out_hbm.at[pl.ds(off, CHG)])
        return 0
    lax.fori_loop(0, NCHUNKG, body, 0)


@functools.partial(
    pl.kernel,
    out_type=jax.ShapeDtypeStruct((NC, N, UNITS), jnp.float32),
    mesh=_sc_mesh,
    scratch_types=dict(
        sidx_v=pltpu.VMEM((CH,), jnp.int32),
        didx_v=pltpu.VMEM((CH,), jnp.int32),
        rows_v=pltpu.VMEM((CH, UNITS), jnp.float32),
        acc=pltpu.VMEM_SHARED((N, UNITS), jnp.float32),
        sem=pltpu.SemaphoreType.DMA,
    ),
)
def _sc_gather_segsum(table_hbm, src_hbm, dst_hbm, out_hbm,
                      sidx_v, didx_v, rows_v, acc, sem):
    cid = lax.axis_index("c")
    sid = lax.axis_index("s")
    wid = sid * NC + cid
    _zero_acc(acc, rows_v, sid)
    plsc.subcore_barrier()
    base = wid * EPW

    def body(k, _):
        off = pl.multiple_of(base + k * CH, 8)
        pltpu.sync_copy(src_hbm.at[pl.ds(off, CH)], sidx_v)
        pltpu.sync_copy(dst_hbm.at[pl.ds(off, CH)], didx_v)
        pltpu.async_copy(table_hbm.at[sidx_v], rows_v, sem).wait()
        pltpu.sync_copy(rows_v, acc.at[didx_v], add=True)
        return 0
    lax.fori_loop(0, NCHUNK, body, 0)
    plsc.subcore_barrier()
    _acc_to_out(acc, rows_v, sid, out_hbm.at[cid])


def _mm_body(x_ref, w_ref, o_ref):
    o_ref[...] = jnp.dot(x_ref[...], w_ref[...],
                         preferred_element_type=jnp.float32)


def _mm2_sum_body(a_ref, p0_ref, p1_ref, wa_ref, wb_ref, o_ref):
    o_ref[...] = (jnp.dot(a_ref[...], wa_ref[...], preferred_element_type=jnp.float32)
                  + jnp.dot(p0_ref[...] + p1_ref[...], wb_ref[...],
                            preferred_element_type=jnp.float32))


def _sum2_mm_bias_body(p0_ref, p1_ref, w_ref, b_ref, o_ref):
    o_ref[...] = jnp.dot(p0_ref[...] + p1_ref[...], w_ref[...],
                         preferred_element_type=jnp.float32) + b_ref[...]


def _init_body(g_ref, ef_ref, wb_ref, b_ref, o_ref):
    o_ref[...] = jax.nn.relu(
        g_ref[...] + jnp.dot(ef_ref[...], wb_ref[...],
                             preferred_element_type=jnp.float32) + b_ref[...])


def _edge_step_fused_body(h0_ref, g_ref, hwrev_ref, wnext_ref, h_ref, hw_ref):
    h = jax.nn.relu(h0_ref[...] + g_ref[...] - hwrev_ref[...])
    h_ref[...] = h
    hw_ref[...] = jnp.dot(h, wnext_ref[...], preferred_element_type=jnp.float32)


def _edge_step_last_body(h0_ref, g_ref, hwrev_ref, h_ref):
    h_ref[...] = jax.nn.relu(h0_ref[...] + g_ref[...] - hwrev_ref[...])


def _node_step_fused_body(v_ref, a0_ref, a1_ref, b_ref, s_ref, wnext_ref, o_ref):
    x = jax.nn.relu(s_ref[0, 0] * v_ref[...] + a0_ref[...] + a1_ref[...] + b_ref[...])
    o_ref[...] = jnp.dot(x, wnext_ref[...], preferred_element_type=jnp.float32)


def _node_step_last_body(v_ref, a0_ref, a1_ref, b_ref, s_ref, o_ref):
    o_ref[...] = jax.nn.relu(
        s_ref[0, 0] * v_ref[...] + a0_ref[...] + a1_ref[...] + b_ref[...])


def _rowblk(B, ncols):
    return pl.BlockSpec((B, ncols), lambda i: (i, 0))


def _full(shape):
    return pl.BlockSpec(shape, lambda i: tuple(0 for _ in shape))


def _mm(x, w, B):
    M, K = x.shape
    Nout = w.shape[1]
    return pl.pallas_call(
        _mm_body,
        grid=(M // B,),
        in_specs=[_rowblk(B, K), _full((K, Nout))],
        out_specs=_rowblk(B, Nout),
        out_shape=jax.ShapeDtypeStruct((M, Nout), jnp.float32),
    )(x, w)


def _mm2_sum(a, p0, p1, wa, wb, B):
    M = a.shape[0]
    Nout = wa.shape[1]
    return pl.pallas_call(
        _mm2_sum_body,
        grid=(M // B,),
        in_specs=[_rowblk(B, a.shape[1]), _rowblk(B, p0.shape[1]),
                  _rowblk(B, p1.shape[1]), _full(wa.shape), _full(wb.shape)],
        out_specs=_rowblk(B, Nout),
        out_shape=jax.ShapeDtypeStruct((M, Nout), jnp.float32),
    )(a, p0, p1, wa, wb)


def _sum2_mm_bias(p0, p1, w, b, B):
    M, K = p0.shape
    Nout = w.shape[1]
    return pl.pallas_call(
        _sum2_mm_bias_body,
        grid=(M // B,),
        in_specs=[_rowblk(B, K), _rowblk(B, K), _full((K, Nout)), _full((1, Nout))],
        out_specs=_rowblk(B, Nout),
        out_shape=jax.ShapeDtypeStruct((M, Nout), jnp.float32),
    )(p0, p1, w, b.reshape(1, -1))


def _edge_init(g, ef, wb, b):
    return pl.pallas_call(
        _init_body,
        grid=(EG,),
        in_specs=[_rowblk(EB, UNITS), _rowblk(EB, DE),
                  _full((DE, UNITS)), _full((1, UNITS))],
        out_specs=_rowblk(EB, UNITS),
        out_shape=jax.ShapeDtypeStruct((E, UNITS), jnp.float32),
    )(g, ef, wb, b.reshape(1, -1))


def _rev_spec():
    return pl.BlockSpec((EB, UNITS), lambda i: ((i + EG // 2) % EG, 0))


def _edge_step_fused(h0, g, hw, wnext):
    return pl.pallas_call(
        _edge_step_fused_body,
        grid=(EG,),
        in_specs=[_rowblk(EB, UNITS), _rowblk(EB, UNITS), _rev_spec(),
                  _full((UNITS, UNITS))],
        out_specs=[_rowblk(EB, UNITS), _rowblk(EB, UNITS)],
        out_shape=[jax.ShapeDtypeStruct((E, UNITS), jnp.float32),
                   jax.ShapeDtypeStruct((E, UNITS), jnp.float32)],
    )(h0, g, hw, wnext)


def _edge_step_last(h0, g, hw):
    return pl.pallas_call(
        _edge_step_last_body,
        grid=(EG,),
        in_specs=[_rowblk(EB, UNITS), _rowblk(EB, UNITS), _rev_spec()],
        out_specs=_rowblk(EB, UNITS),
        out_shape=jax.ShapeDtypeStruct((E, UNITS), jnp.float32),
    )(h0, g, hw)


def _node_step(v, a0, a1, b, scale, wnext):
    if wnext is None:
        body, extra_in, extra_specs = _node_step_last_body, (), ()
    else:
        body = _node_step_fused_body
        extra_in = (wnext,)
        extra_specs = (_full((UNITS, UNITS)),)
    return pl.pallas_call(
        body,
        grid=(NG,),
        in_specs=[_rowblk(NB, UNITS), _rowblk(NB, UNITS), _rowblk(NB, UNITS),
                  _full((1, UNITS)), _full((1, 1))] + list(extra_specs),
        out_specs=_rowblk(NB, UNITS),
        out_shape=jax.ShapeDtypeStruct((N, UNITS), jnp.float32),
    )(v, a0, a1, b.reshape(1, -1), scale.reshape(1, 1), *extra_in)


def kernel(node_feature, edge_feature, edge_index, W_init, b_init,
           W_edge, b_edge, W_gin0, b_gin0, W_gin, b_gin, eps):
    src = edge_index[0].astype(jnp.int32)
    dst = edge_index[1].astype(jnp.int32)

    Wa = W_init[:D]
    Wb = W_init[D:]
    nfW = _mm(node_feature, Wa, NB)
    g0 = _sc_gather_rows(nfW, src)
    h0 = _edge_init(g0, edge_feature, Wb, b_init)

    hW = _mm(h0, W_edge[0], EB)
    h = h0
    for t in range(EDGE_STEPS):
        p = _sc_segsum_rows(h, dst)
        u = _sum2_mm_bias(p[0], p[1], W_edge[t], b_edge[t], NB)
        g = _sc_gather_rows(u, src)
        if t < EDGE_STEPS - 1:
            h, hW = _edge_step_fused(h0, g, hW, W_edge[t + 1])
        else:
            h = _edge_step_last(h0, g, hW)

    m = _sc_segsum_rows(h, dst)
    W0a = W_gin0[:D]
    W0b = W_gin0[D:]
    v = _mm2_sum(node_feature, m[0], m[1], W0a, W0b, NB)
    for t in range(NODE_STEPS):
        a = _sc_gather_segsum(v, src, dst)
        b = b_gin0 if t == 0 else b_gin[t - 1]
        wnext = W_gin[t] if t < NODE_STEPS - 1 else None
        v = _node_step(v, a[0], a[1], b, 1.0 + eps[t], wnext)
    return v
